# baseline (speedup 1.0000x reference)
import jax
import jax.numpy as jnp
from jax.experimental import pallas as pl

N_GRAPHS = 128


def _matmul_body(p_ref, w_ref, b_ref, o_ref):
    o_ref[...] = p_ref[...] @ w_ref[...] + b_ref[...]


def _gcn_conv(x, src, dst, n, W, b):
    loops = jnp.arange(n, dtype=src.dtype)
    s = jnp.concatenate([src, loops])
    d = jnp.concatenate([dst, loops])
    xw = x @ W
    deg = jnp.zeros((n,), dtype=x.dtype).at[d].add(1.0)
    dinv = jnp.where(deg > 0, deg ** -0.5, 0.0)
    norm = dinv[s] * dinv[d]
    msg = xw[s] * norm[:, None]
    out = jnp.zeros_like(xw).at[d].add(msg)
    return out + b


def _batch_norm(x, gamma, beta, eps=1e-5):
    mean = jnp.mean(x, axis=0)
    var = jnp.var(x, axis=0)
    return (x - mean) * jax.lax.rsqrt(var + eps) * gamma + beta


def _leaky_relu(x, slope=0.1):
    return jnp.where(x >= 0, x, slope * x)


def kernel(x, edge_index, batch, W1, b1, W2, b2, g1, be1, g2, be2, Wl, bl):
    n = x.shape[0]
    src, dst = edge_index[0], edge_index[1]
    h = _gcn_conv(x, src, dst, n, W1, b1)
    h = _batch_norm(h, g1, be1)
    h = _leaky_relu(h, 0.1)
    h = _gcn_conv(h, src, dst, n, W2, b2)
    h = _batch_norm(h, g2, be2)
    h = _leaky_relu(h, 0.1)
    sums = jax.ops.segment_sum(h, batch, num_segments=N_GRAPHS)
    counts = jax.ops.segment_sum(jnp.ones((n,), dtype=h.dtype), batch, num_segments=N_GRAPHS)
    pooled = sums / jnp.clip(counts, 1.0)[:, None]
    return pl.pallas_call(
        _matmul_body,
        out_shape=jax.ShapeDtypeStruct((N_GRAPHS, Wl.shape[1]), jnp.float32),
    )(pooled, Wl, bl)


# speedup vs baseline: 3.4449x; 3.4449x over previous
import jax
import jax.numpy as jnp
from jax import lax
from jax.experimental import pallas as pl
from jax.experimental.pallas import tpu as pltpu
from jax.experimental.pallas import tpu_sc as plsc

N = 10000
NPAD = 10240
E = 160000
NGR = 128
NC = 2
NS = 16
NW = NC * NS
K = 128
NB = 40
EW = NB * K
EPAD = NW * EW
RPT = NPAD // NS
SPT = NPAD // NW
F32 = jnp.float32
I32 = jnp.int32

_MESH = dict(core_axis_name="c", subcore_axis_name="s", num_cores=NC,
             num_subcores=NS)


def _rsqrt_newton(p):
    i = plsc.bitcast(p, I32)
    i = 0x5F3759DF - lax.shift_right_logical(i, 1)
    y = plsc.bitcast(i, F32)
    for _ in range(3):
        y = y * (1.5 - 0.5 * p * y * y)
    return y


def _deg_body(dstp, maskp, out, deg_sh, dst_v, val_v, zz):
    cid = lax.axis_index("c")
    sid = lax.axis_index("s")
    w = cid * NS + sid

    def _z(i, _):
        zz[pl.ds(i * 16, 16)] = jnp.zeros((16,), F32)
        return 0
    lax.fori_loop(0, RPT // 16, _z, 0)
    pltpu.sync_copy(zz, deg_sh.at[pl.ds(sid * RPT, RPT)])
    plsc.subcore_barrier()

    pltpu.sync_copy(dstp.at[w], dst_v)
    pltpu.sync_copy(maskp.at[w], val_v)

    def _acc(b, _):
        pltpu.sync_copy(val_v.at[b], deg_sh.at[dst_v.at[b]], add=True)
        return 0
    lax.fori_loop(0, NB, _acc, 0)
    plsc.subcore_barrier()

    @pl.when(cid == 0)
    def _():
        pltpu.sync_copy(deg_sh.at[pl.ds(sid * RPT, RPT)],
                        out.at[pl.ds(sid * RPT, RPT)])

    @pl.when(cid == 1)
    def _():
        pltpu.sync_copy(deg_sh.at[pl.ds(sid * RPT, RPT)],
                        out.at[pl.ds(NPAD + sid * RPT, RPT)])


_deg_kernel = pl.kernel(
    _deg_body,
    out_type=jax.ShapeDtypeStruct((NC * NPAD,), F32),
    mesh=plsc.VectorSubcoreMesh(**_MESH),
    compiler_params=pltpu.CompilerParams(needs_layout_passes=False),
    scratch_types=[
        pltpu.VMEM_SHARED((NPAD,), F32),
        pltpu.VMEM((NB, K), I32),
        pltpu.VMEM((NB, K), F32),
        pltpu.VMEM((RPT,), F32),
    ],
)


def _norm_body(deg2, srcp, dstp, maskp, normp,
               dv, tv, src_v, dst_v, msk_v, nrm_v):
    cid = lax.axis_index("c")
    sid = lax.axis_index("s")
    w = cid * NS + sid

    pltpu.sync_copy(deg2.at[pl.ds(0, NPAD)], dv)
    pltpu.sync_copy(deg2.at[pl.ds(NPAD, NPAD)], tv)

    def _sum(i, _):
        s = pl.ds(i * 16, 16)
        dv[s] = dv[s] + tv[s] + 1.0
        return 0
    lax.fori_loop(0, NPAD // 16, _sum, 0)

    pltpu.sync_copy(srcp.at[w], src_v)
    pltpu.sync_copy(dstp.at[w], dst_v)
    pltpu.sync_copy(maskp.at[w], msk_v)

    def _batch(b, _):
        for j in range(K // 16):
            s = pl.ds(j * 16, 16)
            gs = plsc.load_gather(dv, [src_v[b, s]])
            gd = plsc.load_gather(dv, [dst_v[b, s]])
            nrm_v[b, s] = _rsqrt_newton(gs * gd) * msk_v[b, s]
        return 0
    lax.fori_loop(0, NB, _batch, 0)
    pltpu.sync_copy(nrm_v, normp.at[w])


_norm_kernel = pl.kernel(
    _norm_body,
    out_type=jax.ShapeDtypeStruct((NW, NB, K), F32),
    mesh=plsc.VectorSubcoreMesh(**_MESH),
    compiler_params=pltpu.CompilerParams(needs_layout_passes=False),
    scratch_types=[
        pltpu.VMEM((NPAD,), F32),
        pltpu.VMEM((NPAD,), F32),
        pltpu.VMEM((NB, K), I32),
        pltpu.VMEM((NB, K), I32),
        pltpu.VMEM((NB, K), F32),
        pltpu.VMEM((NB, K), F32),
    ],
)


ZR = 64


def _conv_body(xw0, xw1, xw2, xw3, srcp, dstp, normp, out0, out1,
               acc, src_v, dst_v, nrm_v, rows_v, zrow):
    cid = lax.axis_index("c")
    sid = lax.axis_index("s")
    w = cid * NS + sid
    tables = (xw0, xw1, xw2, xw3)

    def _z(r, _):
        for t in range(128 // 16):
            zrow[r, pl.ds(t * 16, 16)] = jnp.zeros((16,), F32)
        return 0
    lax.fori_loop(0, ZR, _z, 0)

    pltpu.sync_copy(srcp.at[w], src_v)
    pltpu.sync_copy(dstp.at[w], dst_v)
    pltpu.sync_copy(normp.at[w], nrm_v)

    for c in range(4):
        def _zero(m, _):
            pltpu.sync_copy(zrow, acc.at[pl.ds(sid * RPT + m * ZR, ZR)])
            return 0
        lax.fori_loop(0, RPT // ZR, _zero, 0)
        plsc.subcore_barrier()

        def _edge_batch(b, _):
            pltpu.sync_copy(tables[c].at[src_v.at[b]], rows_v)

            def _scale(k16, _2):
                norm16 = nrm_v[b, pl.ds(k16 * 16, 16)]
                for k in range(16):
                    sc = norm16[k]
                    for j in range(128 // 16):
                        s = pl.ds(j * 16, 16)
                        rows_v[k16 * 16 + k, s] = rows_v[k16 * 16 + k, s] * sc
                return 0
            lax.fori_loop(0, K // 16, _scale, 0)
            pltpu.sync_copy(rows_v, acc.at[dst_v.at[b]], add=True)
            return 0
        lax.fori_loop(0, NB, _edge_batch, 0)
        plsc.subcore_barrier()

        @pl.when(cid == 0)
        def _():
            pltpu.sync_copy(acc.at[pl.ds(sid * RPT, RPT)],
                            out0.at[pl.ds(c * NPAD + sid * RPT, RPT)])

        @pl.when(cid == 1)
        def _():
            pltpu.sync_copy(acc.at[pl.ds(sid * RPT, RPT)],
                            out1.at[pl.ds(c * NPAD + sid * RPT, RPT)])


_conv_kernel = pl.kernel(
    _conv_body,
    out_type=(jax.ShapeDtypeStruct((4 * NPAD, 128), F32),
              jax.ShapeDtypeStruct((4 * NPAD, 128), F32)),
    mesh=plsc.VectorSubcoreMesh(**_MESH),
    compiler_params=pltpu.CompilerParams(needs_layout_passes=False),
    scratch_types=[
        pltpu.VMEM_SHARED((NPAD, 128), F32),
        pltpu.VMEM((NB, K), I32),
        pltpu.VMEM((NB, K), I32),
        pltpu.VMEM((NB, K), F32),
        pltpu.VMEM((K, 128), F32),
        pltpu.VMEM((ZR, 128), F32),
    ],
)


RB1 = 1024


def _mm1_body(x_ref, w_ref, o0, o1, o2, o3):
    xw = jnp.dot(x_ref[...], w_ref[...], preferred_element_type=F32, precision=lax.Precision.HIGHEST)
    for c, o in enumerate((o0, o1, o2, o3)):
        o[...] = xw[:, c * 128:(c + 1) * 128]


def _mm1(xpad, W1):
    outs = tuple(jax.ShapeDtypeStruct((NPAD, 128), F32) for _ in range(4))
    return pl.pallas_call(
        _mm1_body,
        grid=(NPAD // RB1,),
        in_specs=[
            pl.BlockSpec((RB1, 256), lambda i: (i, 0)),
            pl.BlockSpec((256, 512), lambda i: (0, 0)),
        ],
        out_specs=tuple(pl.BlockSpec((RB1, 128), lambda i: (i, 0))
                        for _ in range(4)),
        out_shape=outs,
    )(xpad, W1)


RBS = 512


def _stats_body(p0, p1, x0, x1, x2, x3, sn, o_ref):
    i = pl.program_id(0)
    rows = []
    for c, xc in enumerate((x0, x1, x2, x3)):
        h = p0[c] + p1[c] + xc[...] * sn[...]
        s = jnp.sum(h, axis=0, keepdims=True)
        s2 = jnp.sum(h * h, axis=0, keepdims=True)
        rows.append(jnp.concatenate([s, s2], axis=0)[:, None, :])
    blk = jnp.concatenate(rows, axis=1)

    @pl.when(i == 0)
    def _():
        o_ref[...] = blk

    @pl.when(i != 0)
    def _():
        o_ref[...] += blk


def _stats(p0, p1, xws, sn):
    return pl.pallas_call(
        _stats_body,
        grid=(NPAD // RBS,),
        in_specs=[
            pl.BlockSpec((4, RBS, 128), lambda i: (0, i, 0)),
            pl.BlockSpec((4, RBS, 128), lambda i: (0, i, 0)),
            pl.BlockSpec((RBS, 128), lambda i: (i, 0)),
            pl.BlockSpec((RBS, 128), lambda i: (i, 0)),
            pl.BlockSpec((RBS, 128), lambda i: (i, 0)),
            pl.BlockSpec((RBS, 128), lambda i: (i, 0)),
            pl.BlockSpec((RBS, 1), lambda i: (i, 0)),
        ],
        out_specs=pl.BlockSpec((2, 4, 128), lambda i: (0, 0, 0)),
        out_shape=jax.ShapeDtypeStruct((2, 4, 128), F32),
    )(p0, p1, *xws, sn)


def _ss_body(st, g, be, o_ref):
    mean = st[0] / float(N)
    var = st[1] / float(N) - mean * mean
    rs = lax.rsqrt(var + 1e-5)
    scale = g[...] * rs
    shift = be[...] - mean * scale
    o_ref[...] = jnp.stack([scale, shift], axis=0)


def _scale_shift(stats, gr, ber):
    return pl.pallas_call(
        _ss_body,
        out_shape=jax.ShapeDtypeStruct((2, 4, 128), F32),
    )(stats, gr, ber)


def _sn_body(d_ref, o_ref):
    r = lax.broadcasted_iota(I32, (NPAD // 128, 128), 0)
    cc = lax.broadcasted_iota(I32, (NPAD // 128, 128), 1)
    valid = (r * 128 + cc) < N
    o_ref[...] = jnp.where(valid, 1.0 / (d_ref[0] + d_ref[1] + 1.0), 0.0)


def _selfnorm(deg2r):
    return pl.pallas_call(
        _sn_body,
        out_shape=jax.ShapeDtypeStruct((NPAD // 128, 128), F32),
    )(deg2r)

RB2 = 512


def _fused_mm2_body(p0, p1, x0, x1, x2, x3, sn, ss, w2, o0, o1, o2, o3):
    outs = (o0, o1, o2, o3)
    r = None
    for c, xc in enumerate((x0, x1, x2, x3)):
        h = p0[c] + p1[c] + xc[...] * sn[...]
        h = h * ss[0, c][None, :] + ss[1, c][None, :]
        h = jnp.where(h >= 0, h, 0.1 * h)
        d = jnp.dot(h, w2[c], preferred_element_type=F32, precision=lax.Precision.HIGHEST)
        r = d if r is None else r + d
    for c in range(4):
        outs[c][...] = r[:, c * 128:(c + 1) * 128]


def _fused_mm2(p0, p1, xws, sn, ss, W2r):
    outs = tuple(jax.ShapeDtypeStruct((NPAD, 128), F32) for _ in range(4))
    return pl.pallas_call(
        _fused_mm2_body,
        grid=(NPAD // RB2,),
        in_specs=[
            pl.BlockSpec((4, RB2, 128), lambda i: (0, i, 0)),
            pl.BlockSpec((4, RB2, 128), lambda i: (0, i, 0)),
            pl.BlockSpec((RB2, 128), lambda i: (i, 0)),
            pl.BlockSpec((RB2, 128), lambda i: (i, 0)),
            pl.BlockSpec((RB2, 128), lambda i: (i, 0)),
            pl.BlockSpec((RB2, 128), lambda i: (i, 0)),
            pl.BlockSpec((RB2, 1), lambda i: (i, 0)),
            pl.BlockSpec((2, 4, 128), lambda i: (0, 0, 0)),
            pl.BlockSpec((4, 128, 512), lambda i: (0, 0, 0)),
        ],
        out_specs=tuple(pl.BlockSpec((RB2, 128), lambda i: (i, 0))
                        for _ in range(4)),
        out_shape=outs,
    )(p0, p1, *xws, sn, ss, W2r)


def _pool_body(p0, p1, x0, x1, x2, x3, sn, ss, bt, pooled, counts):
    i = pl.program_id(0)
    gid = lax.broadcasted_iota(I32, (RBS, NGR), 1)
    onehot = jnp.where(bt[...] == gid, 1.0, 0.0).astype(F32)
    blks = []
    for c, xc in enumerate((x0, x1, x2, x3)):
        h = p0[c] + p1[c] + xc[...] * sn[...]
        h = h * ss[0, c][None, :] + ss[1, c][None, :]
        h = jnp.where(h >= 0, h, 0.1 * h)
        blks.append(lax.dot_general(onehot, h, (((0,), (0,)), ((), ())),
                                    preferred_element_type=F32,
                                    precision=lax.Precision.HIGHEST))
    blk = jnp.concatenate(blks, axis=1)
    cnt = lax.dot_general(onehot, jnp.ones((RBS, 1), F32),
                          (((0,), (0,)), ((), ())),
                          preferred_element_type=F32, precision=lax.Precision.HIGHEST)

    @pl.when(i == 0)
    def _():
        pooled[...] = blk
        counts[...] = cnt

    @pl.when(i != 0)
    def _():
        pooled[...] += blk
        counts[...] += cnt


def _pool(p0, p1, xws, sn, ss, batchp):
    return pl.pallas_call(
        _pool_body,
        grid=(NPAD // RBS,),
        in_specs=[
            pl.BlockSpec((4, RBS, 128), lambda i: (0, i, 0)),
            pl.BlockSpec((4, RBS, 128), lambda i: (0, i, 0)),
            pl.BlockSpec((RBS, 128), lambda i: (i, 0)),
            pl.BlockSpec((RBS, 128), lambda i: (i, 0)),
            pl.BlockSpec((RBS, 128), lambda i: (i, 0)),
            pl.BlockSpec((RBS, 128), lambda i: (i, 0)),
            pl.BlockSpec((RBS, 1), lambda i: (i, 0)),
            pl.BlockSpec((2, 4, 128), lambda i: (0, 0, 0)),
            pl.BlockSpec((RBS, 1), lambda i: (i, 0)),
        ],
        out_specs=(pl.BlockSpec((NGR, 512), lambda i: (0, 0)),
                   pl.BlockSpec((NGR, 1), lambda i: (0, 0))),
        out_shape=(jax.ShapeDtypeStruct((NGR, 512), F32),
                   jax.ShapeDtypeStruct((NGR, 1), F32)),
    )(p0, p1, *xws, sn, ss, batchp)


def _final_body(pooled, counts, wl, bl, o_ref):
    inv = 1.0 / jnp.maximum(counts[...], 1.0)
    o_ref[...] = jnp.dot(pooled[...] * inv, wl[...],
                         preferred_element_type=F32, precision=lax.Precision.HIGHEST) + bl[...]


def _final(pooled, counts, Wl, blr):
    return pl.pallas_call(
        _final_body,
        out_shape=jax.ShapeDtypeStruct((NGR, 64), F32),
    )(pooled, counts, Wl, blr)


def kernel(x, edge_index, batch, W1, b1, W2, b2, g1, be1, g2, be2, Wl, bl):
    src = edge_index[0]
    dst = edge_index[1]
    srcp = jnp.pad(src, (0, EPAD - E)).reshape(NW, NB, K)
    dstp = jnp.pad(dst, (0, EPAD - E)).reshape(NW, NB, K)
    maskp = jnp.pad(jnp.ones((E,), F32), (0, EPAD - E)).reshape(NW, NB, K)
    xpad = jnp.pad(x, ((0, NPAD - N), (0, 0)))
    batchp = jnp.pad(batch, (0, NPAD - N),
                     constant_values=NGR).reshape(NPAD, 1)
    W2r = W2.reshape(4, 128, 512)
    g1r, be1r = g1.reshape(4, 128), be1.reshape(4, 128)
    g2r, be2r = g2.reshape(4, 128), be2.reshape(4, 128)
    blr = bl.reshape(1, 64)

    deg2 = _deg_kernel(dstp, maskp)
    normp = _norm_kernel(deg2, srcp, dstp, maskp)
    sn = _selfnorm(deg2.reshape(2, NPAD // 128, 128)).reshape(NPAD, 1)

    xw1 = _mm1(xpad, W1)
    m0, m1 = _conv_kernel(*xw1, srcp, dstp, normp)
    p0 = m0.reshape(4, NPAD, 128)
    p1 = m1.reshape(4, NPAD, 128)

    st1 = _stats(p0, p1, xw1, sn)
    ss1 = _scale_shift(st1, g1r, be1r)

    xw2 = _fused_mm2(p0, p1, xw1, sn, ss1, W2r)
    n0, n1 = _conv_kernel(*xw2, srcp, dstp, normp)
    q0 = n0.reshape(4, NPAD, 128)
    q1 = n1.reshape(4, NPAD, 128)

    st2 = _stats(q0, q1, xw2, sn)
    ss2 = _scale_shift(st2, g2r, be2r)

    pooled, counts = _pool(q0, q1, xw2, sn, ss2, batchp)
    return _final(pooled, counts, Wl, blr)


# speedup vs baseline: 3.8655x; 1.1221x over previous
import jax
import jax.numpy as jnp
from jax import lax
from jax.experimental import pallas as pl
from jax.experimental.pallas import tpu as pltpu
from jax.experimental.pallas import tpu_sc as plsc

N = 10000
NPAD = 10240
E = 160000
NGR = 128
NC = 2
NS = 16
NW = NC * NS
K = 64
NB = 80
EW = NB * K
EPAD = NW * EW
RPT = NPAD // NS
SPT = NPAD // NW
F32 = jnp.float32
I32 = jnp.int32

_MESH = dict(core_axis_name="c", subcore_axis_name="s", num_cores=NC,
             num_subcores=NS)


def _rsqrt_newton(p):
    i = plsc.bitcast(p, I32)
    i = 0x5F3759DF - lax.shift_right_logical(i, 1)
    y = plsc.bitcast(i, F32)
    for _ in range(3):
        y = y * (1.5 - 0.5 * p * y * y)
    return y


def _deg_body(dstp, maskp, out, deg_sh, dst_v, val_v, zz):
    cid = lax.axis_index("c")
    sid = lax.axis_index("s")
    w = cid * NS + sid

    def _z(i, _):
        zz[pl.ds(i * 16, 16)] = jnp.zeros((16,), F32)
        return 0
    lax.fori_loop(0, RPT // 16, _z, 0)
    pltpu.sync_copy(zz, deg_sh.at[pl.ds(sid * RPT, RPT)])
    plsc.subcore_barrier()

    pltpu.sync_copy(dstp.at[w], dst_v)
    pltpu.sync_copy(maskp.at[w], val_v)

    def _acc(b, _):
        pltpu.sync_copy(val_v.at[b], deg_sh.at[dst_v.at[b]], add=True)
        return 0
    lax.fori_loop(0, NB, _acc, 0)
    plsc.subcore_barrier()

    @pl.when(cid == 0)
    def _():
        pltpu.sync_copy(deg_sh.at[pl.ds(sid * RPT, RPT)],
                        out.at[pl.ds(sid * RPT, RPT)])

    @pl.when(cid == 1)
    def _():
        pltpu.sync_copy(deg_sh.at[pl.ds(sid * RPT, RPT)],
                        out.at[pl.ds(NPAD + sid * RPT, RPT)])


_deg_kernel = pl.kernel(
    _deg_body,
    out_type=jax.ShapeDtypeStruct((NC * NPAD,), F32),
    mesh=plsc.VectorSubcoreMesh(**_MESH),
    compiler_params=pltpu.CompilerParams(needs_layout_passes=False),
    scratch_types=[
        pltpu.VMEM_SHARED((NPAD,), F32),
        pltpu.VMEM((NB, K), I32),
        pltpu.VMEM((NB, K), F32),
        pltpu.VMEM((RPT,), F32),
    ],
)


def _norm_body(deg2, srcp, dstp, maskp, normp,
               dv, tv, src_v, dst_v, msk_v, nrm_v):
    cid = lax.axis_index("c")
    sid = lax.axis_index("s")
    w = cid * NS + sid

    pltpu.sync_copy(deg2.at[pl.ds(0, NPAD)], dv)
    pltpu.sync_copy(deg2.at[pl.ds(NPAD, NPAD)], tv)

    def _sum(i, _):
        s = pl.ds(i * 16, 16)
        dv[s] = dv[s] + tv[s] + 1.0
        return 0
    lax.fori_loop(0, NPAD // 16, _sum, 0)

    pltpu.sync_copy(srcp.at[w], src_v)
    pltpu.sync_copy(dstp.at[w], dst_v)
    pltpu.sync_copy(maskp.at[w], msk_v)

    def _batch(b, _):
        for j in range(K // 16):
            s = pl.ds(j * 16, 16)
            gs = plsc.load_gather(dv, [src_v[b, s]])
            gd = plsc.load_gather(dv, [dst_v[b, s]])
            nrm_v[b, s] = _rsqrt_newton(gs * gd) * msk_v[b, s]
        return 0
    lax.fori_loop(0, NB, _batch, 0)
    pltpu.sync_copy(nrm_v, normp.at[w])


_norm_kernel = pl.kernel(
    _norm_body,
    out_type=jax.ShapeDtypeStruct((NW, NB, K), F32),
    mesh=plsc.VectorSubcoreMesh(**_MESH),
    compiler_params=pltpu.CompilerParams(needs_layout_passes=False),
    scratch_types=[
        pltpu.VMEM((NPAD,), F32),
        pltpu.VMEM((NPAD,), F32),
        pltpu.VMEM((NB, K), I32),
        pltpu.VMEM((NB, K), I32),
        pltpu.VMEM((NB, K), F32),
        pltpu.VMEM((NB, K), F32),
    ],
)


def _conv_body(xw0, xw1, xw2, xw3, srcp, dstp, normp, zeros_h, out0, out1,
               acc, src_v, dst_v, nrm_v, r0, r1, r2,
               g0, g1, g2, s0, s1, s2):
    cid = lax.axis_index("c")
    sid = lax.axis_index("s")
    w = cid * NS + sid
    tables = (xw0, xw1, xw2, xw3)
    rows = (r0, r1, r2)
    gsem = (g0, g1, g2)
    ssem = (s0, s1, s2)

    pltpu.sync_copy(srcp.at[w], src_v)
    pltpu.sync_copy(dstp.at[w], dst_v)
    pltpu.sync_copy(normp.at[w], nrm_v)

    def _scale(buf, b):
        def body(k16, _):
            norm16 = nrm_v[b, pl.ds(k16 * 16, 16)]
            for k in range(16):
                sc = norm16[k]
                for j in range(128 // 16):
                    s = pl.ds(j * 16, 16)
                    buf[k16 * 16 + k, s] = buf[k16 * 16 + k, s] * sc
            return 0
        lax.fori_loop(0, K // 16, body, 0)

    for c in range(4):
        pltpu.sync_copy(zeros_h, acc.at[pl.ds(sid * RPT, RPT)])
        plsc.subcore_barrier()

        tbl = tables[c]

        def _step(b, cur, wait_free, guard_gather):
            nxt2 = (cur + 2) % 3
            pltpu.make_async_copy(tbl.at[src_v.at[b]],
                                  rows[cur], gsem[cur]).wait()
            _scale(rows[cur], b)
            if wait_free:
                pltpu.make_async_copy(rows[nxt2], acc.at[dst_v.at[b]],
                                      ssem[nxt2]).wait()
            if guard_gather:
                @pl.when(b + 2 < NB)
                def _():
                    pltpu.async_copy(tbl.at[src_v.at[b + 2]],
                                     rows[nxt2], gsem[nxt2])
            else:
                pltpu.async_copy(tbl.at[src_v.at[b + 2]],
                                 rows[nxt2], gsem[nxt2])
            pltpu.async_copy(rows[cur], acc.at[dst_v.at[b]],
                             ssem[cur], add=True)

        pltpu.async_copy(tbl.at[src_v.at[0]], rows[0], gsem[0])
        pltpu.async_copy(tbl.at[src_v.at[1]], rows[1], gsem[1])
        _step(0, 0, wait_free=False, guard_gather=False)
        _step(1, 1, wait_free=True, guard_gather=False)

        def _loop(b, _):
            @pl.when(b % 3 == 0)
            def _():
                _step(b, 0, wait_free=True, guard_gather=True)

            @pl.when(b % 3 == 1)
            def _():
                _step(b, 1, wait_free=True, guard_gather=True)

            @pl.when(b % 3 == 2)
            def _():
                _step(b, 2, wait_free=True, guard_gather=True)
            return 0
        lax.fori_loop(2, NB, _loop, 0)

        tail = (NB - 1) % 3
        pltpu.make_async_copy(rows[tail], acc.at[dst_v.at[NB - 1]],
                              ssem[tail]).wait()
        plsc.subcore_barrier()

        @pl.when(cid == 0)
        def _():
            pltpu.sync_copy(acc.at[pl.ds(sid * RPT, RPT)],
                            out0.at[pl.ds(c * NPAD + sid * RPT, RPT)])

        @pl.when(cid == 1)
        def _():
            pltpu.sync_copy(acc.at[pl.ds(sid * RPT, RPT)],
                            out1.at[pl.ds(c * NPAD + sid * RPT, RPT)])


_conv_kernel = pl.kernel(
    _conv_body,
    out_type=(jax.ShapeDtypeStruct((4 * NPAD, 128), F32),
              jax.ShapeDtypeStruct((4 * NPAD, 128), F32)),
    mesh=plsc.VectorSubcoreMesh(**_MESH),
    compiler_params=pltpu.CompilerParams(needs_layout_passes=False, use_tc_tiling_on_sc=False),
    scratch_types=[
        pltpu.VMEM_SHARED((NPAD, 128), F32),
        pltpu.VMEM((NB, K), I32),
        pltpu.VMEM((NB, K), I32),
        pltpu.VMEM((NB, K), F32),
        pltpu.VMEM((K, 128), F32),
        pltpu.VMEM((K, 128), F32),
        pltpu.VMEM((K, 128), F32),
        pltpu.SemaphoreType.DMA,
        pltpu.SemaphoreType.DMA,
        pltpu.SemaphoreType.DMA,
        pltpu.SemaphoreType.DMA,
        pltpu.SemaphoreType.DMA,
        pltpu.SemaphoreType.DMA,
    ],
)


RB1 = 1024


def _mm1_body(x_ref, w_ref, o0, o1, o2, o3):
    xw = jnp.dot(x_ref[...], w_ref[...], preferred_element_type=F32, precision=lax.Precision.HIGHEST)
    for c, o in enumerate((o0, o1, o2, o3)):
        o[...] = xw[:, c * 128:(c + 1) * 128]


def _mm1(xpad, W1):
    outs = tuple(jax.ShapeDtypeStruct((NPAD, 128), F32) for _ in range(4))
    return pl.pallas_call(
        _mm1_body,
        grid=(NPAD // RB1,),
        in_specs=[
            pl.BlockSpec((RB1, 256), lambda i: (i, 0)),
            pl.BlockSpec((256, 512), lambda i: (0, 0)),
        ],
        out_specs=tuple(pl.BlockSpec((RB1, 128), lambda i: (i, 0))
                        for _ in range(4)),
        out_shape=outs,
    )(xpad, W1)


RBS = 512


def _stats_body(p0, p1, x0, x1, x2, x3, sn, o_ref):
    i = pl.program_id(0)
    rows = []
    for c, xc in enumerate((x0, x1, x2, x3)):
        h = p0[c] + p1[c] + xc[...] * sn[...]
        s = jnp.sum(h, axis=0, keepdims=True)
        s2 = jnp.sum(h * h, axis=0, keepdims=True)
        rows.append(jnp.concatenate([s, s2], axis=0)[:, None, :])
    blk = jnp.concatenate(rows, axis=1)

    @pl.when(i == 0)
    def _():
        o_ref[...] = blk

    @pl.when(i != 0)
    def _():
        o_ref[...] += blk


def _stats(p0, p1, xws, sn):
    return pl.pallas_call(
        _stats_body,
        grid=(NPAD // RBS,),
        in_specs=[
            pl.BlockSpec((4, RBS, 128), lambda i: (0, i, 0)),
            pl.BlockSpec((4, RBS, 128), lambda i: (0, i, 0)),
            pl.BlockSpec((RBS, 128), lambda i: (i, 0)),
            pl.BlockSpec((RBS, 128), lambda i: (i, 0)),
            pl.BlockSpec((RBS, 128), lambda i: (i, 0)),
            pl.BlockSpec((RBS, 128), lambda i: (i, 0)),
            pl.BlockSpec((RBS, 1), lambda i: (i, 0)),
        ],
        out_specs=pl.BlockSpec((2, 4, 128), lambda i: (0, 0, 0)),
        out_shape=jax.ShapeDtypeStruct((2, 4, 128), F32),
    )(p0, p1, *xws, sn)


def _ss_body(st, g, be, o_ref):
    mean = st[0] / float(N)
    var = st[1] / float(N) - mean * mean
    rs = lax.rsqrt(var + 1e-5)
    scale = g[...] * rs
    shift = be[...] - mean * scale
    o_ref[...] = jnp.stack([scale, shift], axis=0)


def _scale_shift(stats, gr, ber):
    return pl.pallas_call(
        _ss_body,
        out_shape=jax.ShapeDtypeStruct((2, 4, 128), F32),
    )(stats, gr, ber)


def _sn_body(d_ref, o_ref):
    r = lax.broadcasted_iota(I32, (NPAD // 128, 128), 0)
    cc = lax.broadcasted_iota(I32, (NPAD // 128, 128), 1)
    valid = (r * 128 + cc) < N
    o_ref[...] = jnp.where(valid, 1.0 / (d_ref[0] + d_ref[1] + 1.0), 0.0)


def _selfnorm(deg2r):
    return pl.pallas_call(
        _sn_body,
        out_shape=jax.ShapeDtypeStruct((NPAD // 128, 128), F32),
    )(deg2r)

RB2 = 512


def _fused_mm2_body(p0, p1, x0, x1, x2, x3, sn, ss, w2, o0, o1, o2, o3):
    outs = (o0, o1, o2, o3)
    r = None
    for c, xc in enumerate((x0, x1, x2, x3)):
        h = p0[c] + p1[c] + xc[...] * sn[...]
        h = h * ss[0, c][None, :] + ss[1, c][None, :]
        h = jnp.where(h >= 0, h, 0.1 * h)
        d = jnp.dot(h, w2[c], preferred_element_type=F32, precision=lax.Precision.HIGHEST)
        r = d if r is None else r + d
    for c in range(4):
        outs[c][...] = r[:, c * 128:(c + 1) * 128]


def _fused_mm2(p0, p1, xws, sn, ss, W2r):
    outs = tuple(jax.ShapeDtypeStruct((NPAD, 128), F32) for _ in range(4))
    return pl.pallas_call(
        _fused_mm2_body,
        grid=(NPAD // RB2,),
        in_specs=[
            pl.BlockSpec((4, RB2, 128), lambda i: (0, i, 0)),
            pl.BlockSpec((4, RB2, 128), lambda i: (0, i, 0)),
            pl.BlockSpec((RB2, 128), lambda i: (i, 0)),
            pl.BlockSpec((RB2, 128), lambda i: (i, 0)),
            pl.BlockSpec((RB2, 128), lambda i: (i, 0)),
            pl.BlockSpec((RB2, 128), lambda i: (i, 0)),
            pl.BlockSpec((RB2, 1), lambda i: (i, 0)),
            pl.BlockSpec((2, 4, 128), lambda i: (0, 0, 0)),
            pl.BlockSpec((4, 128, 512), lambda i: (0, 0, 0)),
        ],
        out_specs=tuple(pl.BlockSpec((RB2, 128), lambda i: (i, 0))
                        for _ in range(4)),
        out_shape=outs,
    )(p0, p1, *xws, sn, ss, W2r)


def _pool_body(p0, p1, x0, x1, x2, x3, sn, ss, bt, pooled, counts):
    i = pl.program_id(0)
    gid = lax.broadcasted_iota(I32, (RBS, NGR), 1)
    onehot = jnp.where(bt[...] == gid, 1.0, 0.0).astype(F32)
    blks = []
    for c, xc in enumerate((x0, x1, x2, x3)):
        h = p0[c] + p1[c] + xc[...] * sn[...]
        h = h * ss[0, c][None, :] + ss[1, c][None, :]
        h = jnp.where(h >= 0, h, 0.1 * h)
        blks.append(lax.dot_general(onehot, h, (((0,), (0,)), ((), ())),
                                    preferred_element_type=F32,
                                    precision=lax.Precision.HIGHEST))
    blk = jnp.concatenate(blks, axis=1)
    cnt = lax.dot_general(onehot, jnp.ones((RBS, 1), F32),
                          (((0,), (0,)), ((), ())),
                          preferred_element_type=F32, precision=lax.Precision.HIGHEST)

    @pl.when(i == 0)
    def _():
        pooled[...] = blk
        counts[...] = cnt

    @pl.when(i != 0)
    def _():
        pooled[...] += blk
        counts[...] += cnt


def _pool(p0, p1, xws, sn, ss, batchp):
    return pl.pallas_call(
        _pool_body,
        grid=(NPAD // RBS,),
        in_specs=[
            pl.BlockSpec((4, RBS, 128), lambda i: (0, i, 0)),
            pl.BlockSpec((4, RBS, 128), lambda i: (0, i, 0)),
            pl.BlockSpec((RBS, 128), lambda i: (i, 0)),
            pl.BlockSpec((RBS, 128), lambda i: (i, 0)),
            pl.BlockSpec((RBS, 128), lambda i: (i, 0)),
            pl.BlockSpec((RBS, 128), lambda i: (i, 0)),
            pl.BlockSpec((RBS, 1), lambda i: (i, 0)),
            pl.BlockSpec((2, 4, 128), lambda i: (0, 0, 0)),
            pl.BlockSpec((RBS, 1), lambda i: (i, 0)),
        ],
        out_specs=(pl.BlockSpec((NGR, 512), lambda i: (0, 0)),
                   pl.BlockSpec((NGR, 1), lambda i: (0, 0))),
        out_shape=(jax.ShapeDtypeStruct((NGR, 512), F32),
                   jax.ShapeDtypeStruct((NGR, 1), F32)),
    )(p0, p1, *xws, sn, ss, batchp)


def _final_body(pooled, counts, wl, bl, o_ref):
    inv = 1.0 / jnp.maximum(counts[...], 1.0)
    o_ref[...] = jnp.dot(pooled[...] * inv, wl[...],
                         preferred_element_type=F32, precision=lax.Precision.HIGHEST) + bl[...]


def _final(pooled, counts, Wl, blr):
    return pl.pallas_call(
        _final_body,
        out_shape=jax.ShapeDtypeStruct((NGR, 64), F32),
    )(pooled, counts, Wl, blr)


def kernel(x, edge_index, batch, W1, b1, W2, b2, g1, be1, g2, be2, Wl, bl):
    src = edge_index[0]
    dst = edge_index[1]
    srcp = jnp.pad(src, (0, EPAD - E)).reshape(NW, NB, K)
    dstp = jnp.pad(dst, (0, EPAD - E)).reshape(NW, NB, K)
    maskp = jnp.pad(jnp.ones((E,), F32), (0, EPAD - E)).reshape(NW, NB, K)
    xpad = jnp.pad(x, ((0, NPAD - N), (0, 0)))
    batchp = jnp.pad(batch, (0, NPAD - N),
                     constant_values=NGR).reshape(NPAD, 1)
    W2r = W2.reshape(4, 128, 512)
    g1r, be1r = g1.reshape(4, 128), be1.reshape(4, 128)
    g2r, be2r = g2.reshape(4, 128), be2.reshape(4, 128)
    blr = bl.reshape(1, 64)

    deg2 = _deg_kernel(dstp, maskp)
    normp = _norm_kernel(deg2, srcp, dstp, maskp)
    sn = _selfnorm(deg2.reshape(2, NPAD // 128, 128)).reshape(NPAD, 1)

    xw1 = _mm1(xpad, W1)
    zeros_h = jnp.zeros((RPT, 128), F32)
    m0, m1 = _conv_kernel(*xw1, srcp, dstp, normp, zeros_h)
    p0 = m0.reshape(4, NPAD, 128)
    p1 = m1.reshape(4, NPAD, 128)

    st1 = _stats(p0, p1, xw1, sn)
    ss1 = _scale_shift(st1, g1r, be1r)

    xw2 = _fused_mm2(p0, p1, xw1, sn, ss1, W2r)
    n0, n1 = _conv_kernel(*xw2, srcp, dstp, normp, zeros_h)
    q0 = n0.reshape(4, NPAD, 128)
    q1 = n1.reshape(4, NPAD, 128)

    st2 = _stats(q0, q1, xw2, sn)
    ss2 = _scale_shift(st2, g2r, be2r)

    pooled, counts = _pool(q0, q1, xw2, sn, ss2, batchp)
    return _final(pooled, counts, Wl, blr)
